# baseline (speedup 1.0000x reference)
import functools

import jax
import jax.numpy as jnp
from jax import lax
from jax.experimental import pallas as pl
from jax.experimental.pallas import tpu as pltpu
from jax.experimental.pallas import tpu_sc as plsc

_NUM_SC_CORES = 2
_NUM_SC_SUBCORES = 16
_NUM_WORKERS = _NUM_SC_CORES * _NUM_SC_SUBCORES
_CHUNK = 128


def _sc_gather(x2, idx2, rp, ca, cb):
    n, f = x2.shape
    stage = -(-(max(ca, cb) + 9) // 8) * 8
    mesh = plsc.VectorSubcoreMesh(core_axis_name="c", subcore_axis_name="s")

    @functools.partial(
        pl.kernel,
        mesh=mesh,
        out_type=jax.ShapeDtypeStruct((rp, f), jnp.float32),
        scratch_types=[
            pltpu.VMEM((stage, _CHUNK), jnp.int32),
            pltpu.VMEM((_CHUNK, f), jnp.float32),
            pltpu.VMEM((_CHUNK, f), jnp.float32),
            pltpu.SemaphoreType.DMA,
            pltpu.SemaphoreType.DMA,
        ],
    )
    def gather_kernel(x_hbm, idx_hbm, g_hbm, idx_v, buf0, buf1, sem0, sem1):
        c_ax = lax.axis_index("c")
        s_ax = lax.axis_index("s")
        my_cw = jnp.where(c_ax == 0, ca, cb)
        cbase = jnp.where(
            c_ax == 0, s_ax * ca, _NUM_SC_SUBCORES * ca + s_ax * cb
        )
        sh = lax.rem(cbase, 8)
        base_al = pl.multiple_of(cbase - sh, 8)
        pltpu.sync_copy(idx_hbm.at[pl.ds(base_al, stage)], idx_v)
        pltpu.make_async_copy(x_hbm.at[idx_v.at[sh]], buf0, sem0).start()

        def body(i, carry):
            c0 = 2 * i
            c1 = c0 + 1
            pltpu.make_async_copy(x_hbm.at[idx_v.at[sh + c1]], buf1, sem1).start()
            pltpu.make_async_copy(x_hbm.at[idx_v.at[sh + c0]], buf0, sem0).wait()
            pltpu.sync_copy(buf0, g_hbm.at[pl.ds((cbase + c0) * _CHUNK, _CHUNK)])
            pltpu.make_async_copy(x_hbm.at[idx_v.at[sh + c0 + 2]], buf0, sem0).start()
            pltpu.make_async_copy(x_hbm.at[idx_v.at[sh + c1]], buf1, sem1).wait()
            pltpu.sync_copy(buf1, g_hbm.at[pl.ds((cbase + c1) * _CHUNK, _CHUNK)])
            return carry

        lax.fori_loop(0, my_cw // 2, body, 0)
        pltpu.make_async_copy(x_hbm.at[idx_v.at[sh + my_cw]], buf0, sem0).wait()

    return gather_kernel(x2, idx2)


def _tc_body(g_ref, w1t_ref, b_ref, w2r_ref, bias_ref, o_ref, *, p, k, f, h, out, n,
             row0):
    xn2 = g_ref[...].astype(jnp.float32)
    xn = xn2.reshape(p, k, f)
    xr = xn - xn[:, 0:1, :]
    logits = jnp.dot(
        xr.reshape(p * k, f), w1t_ref[...], preferred_element_type=jnp.float32
    )
    logits = logits + b_ref[0:1, :]
    m = jnp.max(logits, axis=1, keepdims=True)
    e = jnp.exp(logits - m)
    q = e * (1.0 / jnp.sum(e, axis=1, keepdims=True))
    sub = 128
    nblk = (p * k) // sub
    lane_grp = jax.lax.broadcasted_iota(jnp.int32, (sub, sub), 1) // k
    row_grp = jax.lax.broadcasted_iota(jnp.int32, (sub, sub), 0) // k
    diag = lane_grp == row_grp
    fparts = []
    for s_ in range(nblk):
        qs = q[s_ * sub:(s_ + 1) * sub, :]
        qt = jnp.concatenate([qs] * (sub // k), axis=1)
        qbd = jnp.where(diag, qt, 0.0)
        xs = xn2[s_ * sub:(s_ + 1) * sub, :]
        fparts.append(
            jax.lax.dot_general(
                qbd, xs, (((0,), (0,)), ((), ())),
                preferred_element_type=jnp.float32,
            )
        )
    facc = jnp.concatenate(fparts, axis=0).reshape(p, h, f)
    acc = jnp.zeros((p, out), jnp.float32) + bias_ref[0:1, :]
    for hh in range(h):
        acc = acc + jnp.dot(
            facc[:, hh, :], w2r_ref[hh], preferred_element_type=jnp.float32
        )
    acc = jnp.maximum(acc, 0.0)
    row = jax.lax.broadcasted_iota(jnp.int32, (p, 1), 0) + pl.program_id(0) * p + row0
    o_ref[...] = jnp.where(row == n - 1, 0.0, acc)


def _tc_compute(g, w1t, b8, w2r, bias8, nh, p, row0, n):
    k = 16
    f = w1t.shape[0]
    h = w1t.shape[1]
    out = w2r.shape[2]
    body = functools.partial(_tc_body, p=p, k=k, f=f, h=h, out=out, n=n, row0=row0)
    return pl.pallas_call(
        body,
        grid=(nh // p,),
        in_specs=[
            pl.BlockSpec((p * k, f), lambda i: (i, 0)),
            pl.BlockSpec((f, h), lambda i: (0, 0)),
            pl.BlockSpec((8, h), lambda i: (0, 0)),
            pl.BlockSpec((h, f, out), lambda i: (0, 0, 0)),
            pl.BlockSpec((8, out), lambda i: (0, 0)),
        ],
        out_specs=pl.BlockSpec((p, out), lambda i: (i, 0)),
        out_shape=jax.ShapeDtypeStruct((nh, out), jnp.float32),
        compiler_params=pltpu.CompilerParams(dimension_semantics=("parallel",)),
    )(g, w1t, b8, w2r, bias8)


def kernel(x, neighbor_index, mlp_W, mlp_b, mlp_out_W, bias):
    b, n, feats = x.shape
    k = neighbor_index.shape[2]
    h = mlp_W.shape[0]
    out_c = bias.shape[0]

    x2 = x.reshape(n, feats)
    w1t = mlp_W.T
    b8 = jnp.broadcast_to(mlp_b.reshape(1, h), (8, h))
    w2r = mlp_out_W.reshape(h, out_c, feats).transpose(0, 2, 1)
    bias8 = jnp.broadcast_to(bias.reshape(1, out_c), (8, out_c))

    halves = 2
    nh = n // halves
    p = max(d for d in range(8, 257, 8) if nh % d == 0)
    outs = []
    for t in range(halves):
        nit = neighbor_index[0, t * nh:(t + 1) * nh].reshape(-1).astype(jnp.int32)
        r = nh * k
        n_chunks = -(-r // _CHUNK)
        cw = -(-n_chunks // _NUM_WORKERS)
        if cw % 2:
            cw += 1
        padded_chunks = cw * _NUM_WORKERS
        rp = padded_chunks * _CHUNK
        cb = cw // 2
        ca = 2 * cw - cb
        extra = -(-(max(ca, cb) + 9) // 8) * 8
        idx_pad = jnp.zeros(((padded_chunks + extra) * _CHUNK,), jnp.int32).at[:r].set(nit)
        idx2 = idx_pad.reshape(padded_chunks + extra, _CHUNK)
        g = _sc_gather(x2, idx2, rp, ca, cb)
        outs.append(_tc_compute(g, w1t, b8, w2r, bias8, nh, p, t * nh, n))
    out = jnp.concatenate(outs, axis=0)
    return out.reshape(b, n, out_c)


# speedup vs baseline: 1.1360x; 1.1360x over previous
import functools

import jax
import jax.numpy as jnp
from jax import lax
from jax.experimental import pallas as pl
from jax.experimental.pallas import tpu as pltpu
from jax.experimental.pallas import tpu_sc as plsc

_NUM_SC_CORES = 2
_NUM_SC_SUBCORES = 16
_NUM_WORKERS = _NUM_SC_CORES * _NUM_SC_SUBCORES
_CHUNK = 128


def _sc_gather(x2, idx2, rp, ca, cb):
    n, f = x2.shape
    stage = -(-(max(ca, cb) + 9) // 8) * 8
    mesh = plsc.VectorSubcoreMesh(core_axis_name="c", subcore_axis_name="s")

    @functools.partial(
        pl.kernel,
        mesh=mesh,
        out_type=jax.ShapeDtypeStruct((rp, f), jnp.float32),
        scratch_types=[
            pltpu.VMEM((stage, _CHUNK), jnp.int32),
            pltpu.VMEM((_CHUNK, f), jnp.float32),
            pltpu.VMEM((_CHUNK, f), jnp.float32),
            pltpu.SemaphoreType.DMA,
            pltpu.SemaphoreType.DMA,
        ],
    )
    def gather_kernel(x_hbm, idx_hbm, g_hbm, idx_v, buf0, buf1, sem0, sem1):
        c_ax = lax.axis_index("c")
        s_ax = lax.axis_index("s")
        my_cw = jnp.where(c_ax == 0, ca, cb)
        cbase = jnp.where(
            c_ax == 0, s_ax * ca, _NUM_SC_SUBCORES * ca + s_ax * cb
        )
        sh = lax.rem(cbase, 8)
        base_al = pl.multiple_of(cbase - sh, 8)
        pltpu.sync_copy(idx_hbm.at[pl.ds(base_al, stage)], idx_v)
        pltpu.make_async_copy(x_hbm.at[idx_v.at[sh]], buf0, sem0).start()

        def body(i, carry):
            c0 = 2 * i
            c1 = c0 + 1
            pltpu.make_async_copy(x_hbm.at[idx_v.at[sh + c1]], buf1, sem1).start()
            pltpu.make_async_copy(x_hbm.at[idx_v.at[sh + c0]], buf0, sem0).wait()
            pltpu.sync_copy(buf0, g_hbm.at[pl.ds((cbase + c0) * _CHUNK, _CHUNK)])
            pltpu.make_async_copy(x_hbm.at[idx_v.at[sh + c0 + 2]], buf0, sem0).start()
            pltpu.make_async_copy(x_hbm.at[idx_v.at[sh + c1]], buf1, sem1).wait()
            pltpu.sync_copy(buf1, g_hbm.at[pl.ds((cbase + c1) * _CHUNK, _CHUNK)])
            return carry

        lax.fori_loop(0, my_cw // 2, body, 0)
        pltpu.make_async_copy(x_hbm.at[idx_v.at[sh + my_cw]], buf0, sem0).wait()

    return gather_kernel(x2, idx2)


def _tc_body(g_ref, w1t_ref, b_ref, w2r_ref, bias_ref, o_ref, *, p, k, f, h, out, n,
             row0):
    xn2 = g_ref[...].astype(jnp.float32)
    xn = xn2.reshape(p, k, f)
    xr = xn - xn[:, 0:1, :]
    logits = jnp.dot(
        xr.reshape(p * k, f), w1t_ref[...], preferred_element_type=jnp.float32
    )
    logits = logits + b_ref[0:1, :]
    m = jnp.max(logits, axis=1, keepdims=True)
    e = jnp.exp(logits - m)
    q = e * (1.0 / jnp.sum(e, axis=1, keepdims=True))
    sub = 128
    nblk = (p * k) // sub
    lane_grp = jax.lax.broadcasted_iota(jnp.int32, (sub, sub), 1) // k
    row_grp = jax.lax.broadcasted_iota(jnp.int32, (sub, sub), 0) // k
    diag = lane_grp == row_grp
    fparts = []
    for s_ in range(nblk):
        qs = q[s_ * sub:(s_ + 1) * sub, :]
        qt = jnp.concatenate([qs] * (sub // k), axis=1)
        qbd = jnp.where(diag, qt, 0.0)
        xs = xn2[s_ * sub:(s_ + 1) * sub, :]
        fparts.append(
            jax.lax.dot_general(
                qbd, xs, (((0,), (0,)), ((), ())),
                preferred_element_type=jnp.float32,
            )
        )
    facc = jnp.concatenate(fparts, axis=0).reshape(p, h, f)
    acc = jnp.zeros((p, out), jnp.float32) + bias_ref[0:1, :]
    for hh in range(h):
        acc = acc + jnp.dot(
            facc[:, hh, :], w2r_ref[hh], preferred_element_type=jnp.float32
        )
    acc = jnp.maximum(acc, 0.0)
    row = jax.lax.broadcasted_iota(jnp.int32, (p, 1), 0) + pl.program_id(0) * p + row0
    o_ref[...] = jnp.where(row == n - 1, 0.0, acc)


def _tc_compute(g, w1t, b8, w2r, bias8, nh, p, row0, n):
    k = 16
    f = w1t.shape[0]
    h = w1t.shape[1]
    out = w2r.shape[2]
    body = functools.partial(_tc_body, p=p, k=k, f=f, h=h, out=out, n=n, row0=row0)
    return pl.pallas_call(
        body,
        grid=(nh // p,),
        in_specs=[
            pl.BlockSpec((p * k, f), lambda i: (i, 0)),
            pl.BlockSpec((f, h), lambda i: (0, 0)),
            pl.BlockSpec((8, h), lambda i: (0, 0)),
            pl.BlockSpec((h, f, out), lambda i: (0, 0, 0)),
            pl.BlockSpec((8, out), lambda i: (0, 0)),
        ],
        out_specs=pl.BlockSpec((p, out), lambda i: (i, 0)),
        out_shape=jax.ShapeDtypeStruct((nh, out), jnp.float32),
        compiler_params=pltpu.CompilerParams(dimension_semantics=("parallel",)),
    )(g, w1t, b8, w2r, bias8)


def kernel(x, neighbor_index, mlp_W, mlp_b, mlp_out_W, bias):
    b, n, feats = x.shape
    k = neighbor_index.shape[2]
    h = mlp_W.shape[0]
    out_c = bias.shape[0]

    x2 = x.reshape(n, feats)
    w1t = mlp_W.T
    b8 = jnp.broadcast_to(mlp_b.reshape(1, h), (8, h))
    w2r = mlp_out_W.reshape(h, out_c, feats).transpose(0, 2, 1)
    bias8 = jnp.broadcast_to(bias.reshape(1, out_c), (8, out_c))

    halves = 5
    nh = n // halves
    p = max(d for d in range(8, 257, 8) if nh % d == 0)
    outs = []
    for t in range(halves):
        nit = neighbor_index[0, t * nh:(t + 1) * nh].reshape(-1).astype(jnp.int32)
        r = nh * k
        n_chunks = -(-r // _CHUNK)
        cw = -(-n_chunks // _NUM_WORKERS)
        if cw % 2:
            cw += 1
        padded_chunks = cw * _NUM_WORKERS
        rp = padded_chunks * _CHUNK
        ca = cb = cw
        extra = -(-(max(ca, cb) + 9) // 8) * 8
        idx_pad = jnp.zeros(((padded_chunks + extra) * _CHUNK,), jnp.int32).at[:r].set(nit)
        idx2 = idx_pad.reshape(padded_chunks + extra, _CHUNK)
        g = _sc_gather(x2, idx2, rp, ca, cb)
        outs.append(_tc_compute(g, w1t, b8, w2r, bias8, nh, p, t * nh, n))
    out = jnp.concatenate(outs, axis=0)
    return out.reshape(b, n, out_c)


# speedup vs baseline: 1.2222x; 1.0759x over previous
import functools

import jax
import jax.numpy as jnp
from jax import lax
from jax.experimental import pallas as pl
from jax.experimental.pallas import tpu as pltpu
from jax.experimental.pallas import tpu_sc as plsc

_NUM_SC_CORES = 2
_NUM_SC_SUBCORES = 16
_NUM_WORKERS = _NUM_SC_CORES * _NUM_SC_SUBCORES
_CHUNK = 128


def _sc_gather(x2, idx2, rp, ca, cb):
    n, f = x2.shape
    stage = -(-(max(ca, cb) + 9) // 8) * 8
    mesh = plsc.VectorSubcoreMesh(core_axis_name="c", subcore_axis_name="s")

    @functools.partial(
        pl.kernel,
        mesh=mesh,
        out_type=jax.ShapeDtypeStruct((rp, f), jnp.float32),
        scratch_types=[
            pltpu.VMEM((stage, _CHUNK), jnp.int32),
            pltpu.VMEM((_CHUNK, f), jnp.float32),
            pltpu.VMEM((_CHUNK, f), jnp.float32),
            pltpu.VMEM((_CHUNK, f), jnp.float32),
            pltpu.VMEM((_CHUNK, f), jnp.float32),
            pltpu.SemaphoreType.DMA,
            pltpu.SemaphoreType.DMA,
            pltpu.SemaphoreType.DMA,
            pltpu.SemaphoreType.DMA,
            pltpu.SemaphoreType.DMA,
            pltpu.SemaphoreType.DMA,
            pltpu.SemaphoreType.DMA,
            pltpu.SemaphoreType.DMA,
        ],
    )
    def gather_kernel(x_hbm, idx_hbm, g_hbm, idx_v,
                      b0, b1, b2, b3, g0, g1, g2, g3, w0, w1, w2, w3):
        bufs = [b0, b1, b2, b3]
        gsems = [g0, g1, g2, g3]
        wsems = [w0, w1, w2, w3]
        c_ax = lax.axis_index("c")
        s_ax = lax.axis_index("s")
        my_cw = jnp.where(c_ax == 0, ca, cb)
        cbase = jnp.where(
            c_ax == 0, s_ax * ca, _NUM_SC_SUBCORES * ca + s_ax * cb
        )
        sh = lax.rem(cbase, 8)
        base_al = pl.multiple_of(cbase - sh, 8)
        pltpu.sync_copy(idx_hbm.at[pl.ds(base_al, stage)], idx_v)
        for j in range(4):
            pltpu.make_async_copy(x_hbm.at[idx_v.at[sh + j]], bufs[j], gsems[j]).start()

        def body(i, carry):
            base = 4 * i
            for j in range(4):
                c = base + j
                pltpu.make_async_copy(
                    x_hbm.at[idx_v.at[sh + c]], bufs[j], gsems[j]
                ).wait()
                pltpu.make_async_copy(
                    bufs[j], g_hbm.at[pl.ds((cbase + c) * _CHUNK, _CHUNK)], wsems[j]
                ).start()
            for j in range(4):
                c = base + j
                pltpu.make_async_copy(
                    bufs[j], g_hbm.at[pl.ds((cbase + c) * _CHUNK, _CHUNK)], wsems[j]
                ).wait()
                nc = base + 4 + j

                def _fire(j=j, nc=nc):
                    pltpu.make_async_copy(
                        x_hbm.at[idx_v.at[sh + nc]], bufs[j], gsems[j]
                    ).start()

                pl.when(nc < my_cw)(_fire)
            return carry

        lax.fori_loop(0, my_cw // 4, body, 0)

    return gather_kernel(x2, idx2)


def _tc_body(g_ref, w1t_ref, b_ref, w2r_ref, bias_ref, o_ref, *, p, k, f, h, out, n,
             row0):
    xn2 = g_ref[...].astype(jnp.float32)
    xn = xn2.reshape(p, k, f)
    xr = xn - xn[:, 0:1, :]
    logits = jnp.dot(
        xr.reshape(p * k, f), w1t_ref[...], preferred_element_type=jnp.float32
    )
    logits = logits + b_ref[0:1, :]
    m = jnp.max(logits, axis=1, keepdims=True)
    e = jnp.exp(logits - m)
    q = e * (1.0 / jnp.sum(e, axis=1, keepdims=True))
    sub = 128
    nblk = (p * k) // sub
    lane_grp = jax.lax.broadcasted_iota(jnp.int32, (sub, sub), 1) // k
    row_grp = jax.lax.broadcasted_iota(jnp.int32, (sub, sub), 0) // k
    diag = lane_grp == row_grp
    fparts = []
    for s_ in range(nblk):
        qs = q[s_ * sub:(s_ + 1) * sub, :]
        qt = jnp.concatenate([qs] * (sub // k), axis=1)
        qbd = jnp.where(diag, qt, 0.0)
        xs = xn2[s_ * sub:(s_ + 1) * sub, :]
        fparts.append(
            jax.lax.dot_general(
                qbd, xs, (((0,), (0,)), ((), ())),
                preferred_element_type=jnp.float32,
            )
        )
    facc = jnp.concatenate(fparts, axis=0).reshape(p, h, f)
    acc = jnp.zeros((p, out), jnp.float32) + bias_ref[0:1, :]
    for hh in range(h):
        acc = acc + jnp.dot(
            facc[:, hh, :], w2r_ref[hh], preferred_element_type=jnp.float32
        )
    acc = jnp.maximum(acc, 0.0)
    row = jax.lax.broadcasted_iota(jnp.int32, (p, 1), 0) + pl.program_id(0) * p + row0
    o_ref[...] = jnp.where(row == n - 1, 0.0, acc)


def _tc_compute(g, w1t, b8, w2r, bias8, nh, p, row0, n):
    k = 16
    f = w1t.shape[0]
    h = w1t.shape[1]
    out = w2r.shape[2]
    body = functools.partial(_tc_body, p=p, k=k, f=f, h=h, out=out, n=n, row0=row0)
    return pl.pallas_call(
        body,
        grid=(nh // p,),
        in_specs=[
            pl.BlockSpec((p * k, f), lambda i: (i, 0)),
            pl.BlockSpec((f, h), lambda i: (0, 0)),
            pl.BlockSpec((8, h), lambda i: (0, 0)),
            pl.BlockSpec((h, f, out), lambda i: (0, 0, 0)),
            pl.BlockSpec((8, out), lambda i: (0, 0)),
        ],
        out_specs=pl.BlockSpec((p, out), lambda i: (i, 0)),
        out_shape=jax.ShapeDtypeStruct((nh, out), jnp.float32),
        compiler_params=pltpu.CompilerParams(dimension_semantics=("parallel",)),
    )(g, w1t, b8, w2r, bias8)


def kernel(x, neighbor_index, mlp_W, mlp_b, mlp_out_W, bias):
    b, n, feats = x.shape
    k = neighbor_index.shape[2]
    h = mlp_W.shape[0]
    out_c = bias.shape[0]

    x2 = x.reshape(n, feats)
    w1t = mlp_W.T
    b8 = jnp.broadcast_to(mlp_b.reshape(1, h), (8, h))
    w2r = mlp_out_W.reshape(h, out_c, feats).transpose(0, 2, 1)
    bias8 = jnp.broadcast_to(bias.reshape(1, out_c), (8, out_c))

    halves = 5
    nh = n // halves
    p = max(d for d in range(8, 257, 8) if nh % d == 0)
    outs = []
    for t in range(halves):
        nit = neighbor_index[0, t * nh:(t + 1) * nh].reshape(-1).astype(jnp.int32)
        r = nh * k
        n_chunks = -(-r // _CHUNK)
        cw = -(-n_chunks // _NUM_WORKERS)
        if cw % 4:
            cw += 4 - cw % 4
        padded_chunks = cw * _NUM_WORKERS
        rp = padded_chunks * _CHUNK
        ca = cb = cw
        extra = -(-(max(ca, cb) + 9) // 8) * 8
        idx_pad = jnp.zeros(((padded_chunks + extra) * _CHUNK,), jnp.int32).at[:r].set(nit)
        idx2 = idx_pad.reshape(padded_chunks + extra, _CHUNK)
        g = _sc_gather(x2, idx2, rp, ca, cb)
        outs.append(_tc_compute(g, w1t, b8, w2r, bias8, nh, p, t * nh, n))
    out = jnp.concatenate(outs, axis=0)
    return out.reshape(b, n, out_c)
